# Initial kernel scaffold; baseline (speedup 1.0000x reference)
import jax
import jax.numpy as jnp
from jax.experimental import pallas as pl


def kernel(pos, batch, c1w1, c1b1, c1w2, c1b2, c1w3, c1b3, c2w1, c2b1, c2w2, c2b2, c2w3, c2b3, w0, b0, l1w, l1b, l2w, l2b, l3w, l3b):
    raise NotImplementedError("write your pallas kernel here")


# speedup vs baseline: 1.1911x; 1.1911x over previous
import functools

import jax
import jax.numpy as jnp
from jax import lax
from jax.experimental import pallas as pl
from jax.experimental.pallas import tpu as pltpu

_KNOCK = 1e30
_LANE = 128
_P = 1024
_K = 20
_NC = 10
_Q = 128
_VLIM = 48 * 1024 * 1024


def _knn_onehot(x_full, x_q, *, k):
    P, C = x_full.shape
    Q = x_q.shape[0]
    f32 = jnp.float32
    xsq = x_full * x_full
    sq_row = lax.dot_general(jnp.ones((1, C), f32), xsq,
                             (((1,), (1,)), ((), ())),
                             preferred_element_type=f32)
    gram = lax.dot_general(x_q, x_full, (((1,), (1,)), ((), ())),
                           preferred_element_type=f32)
    dist = sq_row - 2.0 * gram
    lane = lax.broadcasted_iota(jnp.int32, (Q, P), 1).astype(f32)
    picks = []
    dwork = dist
    for _ in range(k):
        rmin = jnp.min(dwork, axis=1, keepdims=True)
        first = jnp.min(jnp.where(dwork == rmin, lane, float(P)),
                        axis=1, keepdims=True)
        pick = lane == first
        picks.append(pick.astype(f32))
        dwork = jnp.where(pick, _KNOCK, dwork)
    return jnp.concatenate(picks, axis=0)


def _conv1_kernel(x_ref, wc_ref, wn_ref, b1_ref, w2_ref, b2_ref, w3_ref,
                  b3_ref, o_ref, *, k, q):
    f32 = jnp.float32
    qc = pl.program_id(1)
    start = pl.multiple_of(qc * q, q)
    x_full = x_ref[0]
    x_q = x_ref[0, pl.ds(start, q), :]
    sel = _knn_onehot(x_full, x_q, k=k)
    n_full = jnp.dot(x_full, wn_ref[...], preferred_element_type=f32)
    ac = jnp.dot(x_q, wc_ref[...], preferred_element_type=f32) + b1_ref[...]
    nj = jnp.dot(sel, n_full, preferred_element_type=f32)
    h = (nj.reshape(k, q, -1) + ac[None, :, :]).reshape(k * q, -1)
    h = jnp.maximum(h, 0.0)
    h = jnp.maximum(jnp.dot(h, w2_ref[...], preferred_element_type=f32)
                    + b2_ref[...], 0.0)
    h = jnp.maximum(jnp.dot(h, w3_ref[...], preferred_element_type=f32)
                    + b3_ref[...], 0.0)
    o_ref[0] = jnp.max(h.reshape(k, q, -1), axis=0)


def _conv2_head_kernel(x_ref, wc_ref, wn_ref, b1_ref, w2_ref, b2_ref, w3_ref,
                       b3_ref, w0_ref, b0_ref, l1w_ref, l1b_ref, l2w_ref,
                       l2b_ref, l3w_ref, l3b_ref, o_ref, pool_acc, *, k, q):
    f32 = jnp.float32
    bf = jnp.bfloat16
    qc = pl.program_id(1)
    start = pl.multiple_of(qc * q, q)
    x_full = x_ref[0]
    x_q = x_ref[0, pl.ds(start, q), :]
    sel = _knn_onehot(x_full, x_q, k=k)

    xb = x_full.astype(bf)
    n_full = jnp.dot(xb, wn_ref[...], preferred_element_type=f32).astype(bf)
    ac = jnp.dot(x_q.astype(bf), wc_ref[...],
                 preferred_element_type=f32) + b1_ref[...]
    nj = jnp.dot(sel.astype(bf), n_full, preferred_element_type=f32)
    h = (nj.reshape(k, q, -1) + ac[None, :, :]).reshape(k * q, -1)
    h = jnp.maximum(h, 0.0)
    h = jnp.maximum(jnp.dot(h.astype(bf), w2_ref[...],
                            preferred_element_type=f32) + b2_ref[...], 0.0)
    h = jnp.maximum(jnp.dot(h.astype(bf), w3_ref[...],
                            preferred_element_type=f32) + b3_ref[...], 0.0)
    feat = jnp.max(h.reshape(k, q, -1), axis=0)

    h0 = jnp.maximum(jnp.dot(feat.astype(bf), w0_ref[...],
                             preferred_element_type=f32) + b0_ref[...], 0.0)
    pooled = jnp.max(h0, axis=0, keepdims=True)

    @pl.when(qc == 0)
    def _():
        pool_acc[...] = pooled

    @pl.when(qc != 0)
    def _():
        pool_acc[...] = jnp.maximum(pool_acc[...], pooled)

    @pl.when(qc == pl.num_programs(1) - 1)
    def _():
        p = pool_acc[...]
        h1 = jnp.maximum(jnp.dot(p, l1w_ref[...], preferred_element_type=f32)
                         + l1b_ref[...], 0.0)
        h2 = jnp.maximum(jnp.dot(h1, l2w_ref[...], preferred_element_type=f32)
                         + l2b_ref[...], 0.0)
        logits = jnp.dot(h2, l3w_ref[...], preferred_element_type=f32) \
            + l3b_ref[...]
        m = jnp.max(logits, axis=1, keepdims=True)
        lse = m + jnp.log(jnp.sum(jnp.exp(logits - m), axis=1, keepdims=True))
        o_ref[0] = logits - lse


def _pad_cols(a, width):
    return a if a.shape[1] == width else jnp.pad(a, ((0, 0), (0, width - a.shape[1])))


def _pad_rows(a, height):
    return a if a.shape[0] == height else jnp.pad(a, ((0, height - a.shape[0]), (0, 0)))


def _split_edge_weights(w1, c_in, c_pad, h_pad, dtype):
    w1a, w1b = w1[:c_in], w1[c_in:]
    if c_pad is not None:
        w1a, w1b = _pad_rows(w1a, c_pad), _pad_rows(w1b, c_pad)
    if h_pad is not None:
        w1a, w1b = _pad_cols(w1a, h_pad), _pad_cols(w1b, h_pad)
    return (w1a - w1b).astype(dtype), w1b.astype(dtype)


def kernel(pos, batch, c1w1, c1b1, c1w2, c1b2, c1w3, c1b3,
           c2w1, c2b1, c2w2, c2b2, c2w3, c2b3,
           w0, b0, l1w, l1b, l2w, l2b, l3w, l3b):
    del batch
    f32 = jnp.float32
    bf = jnp.bfloat16
    n, c = pos.shape
    B = n // _P
    x = pos.reshape(B, _P, c)

    wc1, wn1 = _split_edge_weights(c1w1, c, None, _LANE, f32)
    b1_1 = _pad_cols(c1b1, _LANE)
    w2_1 = _pad_cols(_pad_rows(c1w2, _LANE), _LANE)
    b2_1 = _pad_cols(c1b2, _LANE)
    w3_1 = _pad_cols(_pad_rows(c1w3, _LANE), _LANE)
    b3_1 = _pad_cols(c1b3, _LANE)
    conv1_w = (wc1, wn1, b1_1, w2_1, b2_1, w3_1, b3_1)

    grid1 = pltpu.PrefetchScalarGridSpec(
        num_scalar_prefetch=0,
        grid=(B, _P // _Q),
        in_specs=[pl.BlockSpec((1, _P, c), lambda b, qi: (b, 0, 0))]
                 + [pl.BlockSpec(w.shape, lambda b, qi: (0, 0)) for w in conv1_w],
        out_specs=pl.BlockSpec((1, _Q, _LANE), lambda b, qi: (b, qi, 0)),
    )
    x1 = pl.pallas_call(
        functools.partial(_conv1_kernel, k=_K, q=_Q),
        out_shape=jax.ShapeDtypeStruct((B, _P, _LANE), f32),
        grid_spec=grid1,
        compiler_params=pltpu.CompilerParams(
            dimension_semantics=("parallel", "parallel"),
            vmem_limit_bytes=_VLIM),
    )(x, *conv1_w)

    wc2, wn2 = _split_edge_weights(c2w1, c2w1.shape[0] // 2, _LANE, None, bf)
    conv2_w = (wc2, wn2, c2b1, c2w2.astype(bf), c2b2, c2w3.astype(bf), c2b3)
    head_w = (w0.astype(bf), b0, l1w, l1b, l2w, l2b, l3w, l3b)
    weights = conv2_w + head_w

    grid2 = pltpu.PrefetchScalarGridSpec(
        num_scalar_prefetch=0,
        grid=(B, _P // _Q),
        in_specs=[pl.BlockSpec((1, _P, _LANE), lambda b, qi: (b, 0, 0))]
                 + [pl.BlockSpec(w.shape, lambda b, qi: (0, 0)) for w in weights],
        out_specs=pl.BlockSpec((1, 1, _NC), lambda b, qi: (b, 0, 0)),
        scratch_shapes=[pltpu.VMEM((1, w0.shape[1]), f32)],
    )
    out = pl.pallas_call(
        functools.partial(_conv2_head_kernel, k=_K, q=_Q),
        out_shape=jax.ShapeDtypeStruct((B, 1, _NC), f32),
        grid_spec=grid2,
        compiler_params=pltpu.CompilerParams(
            dimension_semantics=("parallel", "arbitrary"),
            vmem_limit_bytes=_VLIM),
    )(x1, *weights)
    return out[:, 0, :]


# speedup vs baseline: 1.6520x; 1.3869x over previous
import functools

import jax
import jax.numpy as jnp
from jax import lax
from jax.experimental import pallas as pl
from jax.experimental.pallas import tpu as pltpu

_KNOCK = 1e30
_LANE = 128
_P = 1024
_K = 20
_NC = 10
_Q = 256
_VLIM = 48 * 1024 * 1024


def _knn_onehot(x_full, x_q, *, k):
    P, C = x_full.shape
    Q = x_q.shape[0]
    f32 = jnp.float32
    xsq = x_full * x_full
    sq_row = lax.dot_general(jnp.ones((1, C), f32), xsq,
                             (((1,), (1,)), ((), ())),
                             preferred_element_type=f32)
    gram = lax.dot_general(x_q, x_full, (((1,), (1,)), ((), ())),
                           preferred_element_type=f32)
    dist = sq_row - 2.0 * gram
    lane = lax.broadcasted_iota(jnp.int32, (Q, P), 1).astype(f32)
    picks = []
    dwork = dist
    for _ in range(k):
        rmin = jnp.min(dwork, axis=1, keepdims=True)
        first = jnp.min(jnp.where(dwork == rmin, lane, float(P)),
                        axis=1, keepdims=True)
        pick = lane == first
        picks.append(pick.astype(f32))
        dwork = jnp.where(pick, _KNOCK, dwork)
    return jnp.concatenate(picks, axis=0)


def _conv1_kernel(x_ref, wc_ref, wn_ref, b1_ref, w2_ref, b2_ref, w3_ref,
                  b3_ref, o_ref, *, k, q):
    f32 = jnp.float32
    qc = pl.program_id(1)
    start = pl.multiple_of(qc * q, q)
    x_full = x_ref[0]
    x_q = x_ref[0, pl.ds(start, q), :]
    sel = _knn_onehot(x_full, x_q, k=k)
    n_full = jnp.dot(x_full, wn_ref[...], preferred_element_type=f32)
    ac = jnp.dot(x_q, wc_ref[...], preferred_element_type=f32) + b1_ref[...]
    nj = jnp.dot(sel, n_full, preferred_element_type=f32)
    h = (nj.reshape(k, q, -1) + ac[None, :, :]).reshape(k * q, -1)
    h = jnp.maximum(h, 0.0)
    h = jnp.maximum(jnp.dot(h, w2_ref[...], preferred_element_type=f32)
                    + b2_ref[...], 0.0)
    h = jnp.maximum(jnp.dot(h, w3_ref[...], preferred_element_type=f32)
                    + b3_ref[...], 0.0)
    o_ref[0] = jnp.max(h.reshape(k, q, -1), axis=0)


def _conv2_head_kernel(x_ref, wc_ref, wn_ref, b1_ref, w2_ref, b2_ref, w3_ref,
                       b3_ref, w0_ref, b0_ref, l1w_ref, l1b_ref, l2w_ref,
                       l2b_ref, l3w_ref, l3b_ref, o_ref, pool_acc, *, k, q):
    f32 = jnp.float32
    bf = jnp.bfloat16
    qc = pl.program_id(1)
    start = pl.multiple_of(qc * q, q)
    x_full = x_ref[0]
    x_q = x_ref[0, pl.ds(start, q), :]
    sel = _knn_onehot(x_full, x_q, k=k)

    xb = x_full.astype(bf)
    n_full = jnp.dot(xb, wn_ref[...], preferred_element_type=f32).astype(bf)
    ac = jnp.dot(x_q.astype(bf), wc_ref[...],
                 preferred_element_type=f32) + b1_ref[...]
    nj = jnp.dot(sel.astype(bf), n_full, preferred_element_type=f32)
    h = (nj.reshape(k, q, -1) + ac[None, :, :]).reshape(k * q, -1)
    h = jnp.maximum(h, 0.0)
    h = jnp.maximum(jnp.dot(h.astype(bf), w2_ref[...],
                            preferred_element_type=f32) + b2_ref[...], 0.0)
    h = jnp.maximum(jnp.dot(h.astype(bf), w3_ref[...],
                            preferred_element_type=f32) + b3_ref[...], 0.0)
    feat = jnp.max(h.reshape(k, q, -1), axis=0)

    h0 = jnp.maximum(jnp.dot(feat.astype(bf), w0_ref[...],
                             preferred_element_type=f32) + b0_ref[...], 0.0)
    pooled = jnp.max(h0, axis=0, keepdims=True)

    @pl.when(qc == 0)
    def _():
        pool_acc[...] = pooled

    @pl.when(qc != 0)
    def _():
        pool_acc[...] = jnp.maximum(pool_acc[...], pooled)

    @pl.when(qc == pl.num_programs(1) - 1)
    def _():
        p = pool_acc[...]
        h1 = jnp.maximum(jnp.dot(p, l1w_ref[...], preferred_element_type=f32)
                         + l1b_ref[...], 0.0)
        h2 = jnp.maximum(jnp.dot(h1, l2w_ref[...], preferred_element_type=f32)
                         + l2b_ref[...], 0.0)
        logits = jnp.dot(h2, l3w_ref[...], preferred_element_type=f32) \
            + l3b_ref[...]
        m = jnp.max(logits, axis=1, keepdims=True)
        lse = m + jnp.log(jnp.sum(jnp.exp(logits - m), axis=1, keepdims=True))
        o_ref[0] = logits - lse


def _pad_cols(a, width):
    return a if a.shape[1] == width else jnp.pad(a, ((0, 0), (0, width - a.shape[1])))


def _pad_rows(a, height):
    return a if a.shape[0] == height else jnp.pad(a, ((0, height - a.shape[0]), (0, 0)))


def _split_edge_weights(w1, c_in, c_pad, h_pad, dtype):
    w1a, w1b = w1[:c_in], w1[c_in:]
    if c_pad is not None:
        w1a, w1b = _pad_rows(w1a, c_pad), _pad_rows(w1b, c_pad)
    if h_pad is not None:
        w1a, w1b = _pad_cols(w1a, h_pad), _pad_cols(w1b, h_pad)
    return (w1a - w1b).astype(dtype), w1b.astype(dtype)


def kernel(pos, batch, c1w1, c1b1, c1w2, c1b2, c1w3, c1b3,
           c2w1, c2b1, c2w2, c2b2, c2w3, c2b3,
           w0, b0, l1w, l1b, l2w, l2b, l3w, l3b):
    del batch
    f32 = jnp.float32
    bf = jnp.bfloat16
    n, c = pos.shape
    B = n // _P
    x = pos.reshape(B, _P, c)

    wc1, wn1 = _split_edge_weights(c1w1, c, None, None, f32)
    conv1_w = (wc1, wn1, c1b1, c1w2, c1b2, c1w3, c1b3)
    h1 = c1w3.shape[1]

    grid1 = pltpu.PrefetchScalarGridSpec(
        num_scalar_prefetch=0,
        grid=(B, _P // _Q),
        in_specs=[pl.BlockSpec((1, _P, c), lambda b, qi: (b, 0, 0))]
                 + [pl.BlockSpec(w.shape, lambda b, qi: (0, 0)) for w in conv1_w],
        out_specs=pl.BlockSpec((1, _Q, h1), lambda b, qi: (b, qi, 0)),
    )
    x1 = pl.pallas_call(
        functools.partial(_conv1_kernel, k=_K, q=_Q),
        out_shape=jax.ShapeDtypeStruct((B, _P, h1), f32),
        grid_spec=grid1,
        compiler_params=pltpu.CompilerParams(
            dimension_semantics=("parallel", "parallel"),
            vmem_limit_bytes=_VLIM),
    )(x, *conv1_w)

    wc2, wn2 = _split_edge_weights(c2w1, c2w1.shape[0] // 2, None, None, bf)
    wc2, wn2 = wc2[:h1], wn2[:h1]
    conv2_w = (wc2, wn2, c2b1, c2w2.astype(bf), c2b2, c2w3.astype(bf), c2b3)
    head_w = (w0.astype(bf), b0, l1w, l1b, l2w, l2b, l3w, l3b)
    weights = conv2_w + head_w

    grid2 = pltpu.PrefetchScalarGridSpec(
        num_scalar_prefetch=0,
        grid=(B, _P // _Q),
        in_specs=[pl.BlockSpec((1, _P, h1), lambda b, qi: (b, 0, 0))]
                 + [pl.BlockSpec(w.shape, lambda b, qi: (0, 0)) for w in weights],
        out_specs=pl.BlockSpec((1, 1, _NC), lambda b, qi: (b, 0, 0)),
        scratch_shapes=[pltpu.VMEM((1, w0.shape[1]), f32)],
    )
    out = pl.pallas_call(
        functools.partial(_conv2_head_kernel, k=_K, q=_Q),
        out_shape=jax.ShapeDtypeStruct((B, 1, _NC), f32),
        grid_spec=grid2,
        compiler_params=pltpu.CompilerParams(
            dimension_semantics=("parallel", "arbitrary"),
            vmem_limit_bytes=_VLIM),
    )(x1, *weights)
    return out[:, 0, :]
